# baseline (speedup 1.0000x reference)
import functools

import jax
import jax.numpy as jnp
from jax import lax
from jax.experimental import pallas as pl
from jax.experimental.pallas import tpu as pltpu
from jax.experimental.pallas import tpu_sc as plsc

N_SLOTS_K = 8
EPS = 1e-12
BIG = 1e9
L = 16
ROWS_PER_TILE = 64
GROUPS = ROWS_PER_TILE // L


def _rsqrt(x):
    i = plsc.bitcast(x, jnp.int32)
    seed = jnp.int32(0x5F3759DF) - lax.shift_right_logical(i, 1)
    y = plsc.bitcast(seed, jnp.float32)
    for _ in range(3):
        y = y * (1.5 - 0.5 * x * y * y)
    return y


def _sc_greedy(features_hbm, out_hbm, ft_v, sel_v, dots_v, cand_v,
               allcand_v, shared_cand, shared_sel):
    n_slots, d = N_SLOTS_K, out_hbm.shape[2]
    nchunk = d // L
    batches_per_core = features_hbm.shape[0] // 2
    core = lax.axis_index("c")
    sid = lax.axis_index("s")
    iota = lax.iota(jnp.int32, L)
    row_base = sid * ROWS_PER_TILE
    zero = jnp.zeros((L,), jnp.float32)
    gidx = [(row_base + g * L + iota).astype(jnp.float32) for g in range(GROUPS)]

    def streaming_pass(sel_chunks):
        def g_body(g, c):
            def j_body(j, dvec):
                off = (g * L + j) * d
                acc = zero
                for k in range(nchunk):
                    v = ft_v[pl.ds(off + k * L, L)]
                    acc = acc + v * (v if sel_chunks is None else sel_chunks[k])
                s = jnp.sum(acc)
                return jnp.where(iota == j, s, dvec)
            dvec = lax.fori_loop(0, L, j_body, zero, unroll=2)
            dots_v[pl.ds(g * L, L)] = dvec
            return c
        lax.fori_loop(0, GROUPS, g_body, 0)
        return [dots_v[pl.ds(g * L, L)] for g in range(GROUPS)]

    def run_batch(bi, carry):
        b = core * batches_per_core + bi
        pltpu.sync_copy(
            features_hbm.at[b, pl.ds(row_base * d, ROWS_PER_TILE * d)], ft_v)
        sal2 = streaming_pass(None)
        inv_sal = [jnp.minimum(_rsqrt(s), 1.0 / EPS) for s in sal2]
        mask = [jnp.ones((L,), jnp.float32) for _ in range(GROUPS)]

        for r in range(n_slots):
            ms = [sal2[g] * (mask[g] * mask[g]) for g in range(GROUPS)]
            mv = ms[0]
            for g in range(1, GROUPS):
                mv = jnp.maximum(mv, ms[g])
            m = jnp.max(mv)
            cidx = jnp.float32(BIG)
            for g in range(GROUPS):
                cidx = jnp.minimum(
                    cidx, jnp.min(jnp.where(ms[g] == m, gidx[g], BIG)))
            salsel = jnp.float32(-BIG)
            for g in range(GROUPS):
                salsel = jnp.maximum(
                    salsel, jnp.max(jnp.where(gidx[g] == cidx, sal2[g], -BIG)))
            cand = jnp.where(iota == 0, m,
                             jnp.where(iota == 1, cidx,
                                       jnp.where(iota == 2, salsel, 0.0)))
            cand_v[...] = cand
            pltpu.sync_copy(cand_v, shared_cand.at[pl.ds(sid * L, L)])
            plsc.subcore_barrier()
            pltpu.sync_copy(shared_cand, allcand_v)
            gm = jnp.float32(-BIG)
            widx = jnp.float32(BIG)
            wsal = jnp.float32(0.0)
            for t in range(L):
                row = allcand_v[pl.ds(t * L, L)]
                vt, it, st = row[0], row[1], row[2]
                better = (vt > gm) | ((vt == gm) & (it < widx))
                gm = jnp.where(better, vt, gm)
                widx = jnp.where(better, it, widx)
                wsal = jnp.where(better, st, wsal)
            widx_i = widx.astype(jnp.int32)
            wtile = widx_i // ROWS_PER_TILE
            @pl.when(wtile == sid)
            def _():
                pltpu.sync_copy(
                    ft_v.at[pl.ds((widx_i - row_base) * d, d)], shared_sel)
            plsc.subcore_barrier()
            pltpu.sync_copy(shared_sel, sel_v)

            @pl.when(sid == 0)
            def _():
                pltpu.sync_copy(sel_v, out_hbm.at[b, r])

            sel_chunks = [sel_v[pl.ds(k * L, L)] for k in range(nchunk)]
            dots = streaming_pass(sel_chunks)
            inv_sel = jnp.minimum(_rsqrt(jnp.full((L,), wsal)), 1.0 / EPS)
            for g in range(GROUPS):
                sim = dots[g] * inv_sal[g] * inv_sel
                mask[g] = mask[g] * (1.0 - jnp.clip(sim, 0.0, 1.0))
        return carry

    lax.fori_loop(0, batches_per_core, run_batch, 0)


LANES = 128
TC_BB = 8


def _tc_rowdot(a_ref, b_idx, other, n, d):
    acc = a_ref[b_idx, :, pl.ds(0, LANES)] * other[:, 0:LANES]
    for k in range(1, d // LANES):
        acc = acc + (a_ref[b_idx, :, pl.ds(k * LANES, LANES)]
                     * other[:, k * LANES:(k + 1) * LANES])
    return jnp.sum(acc, axis=1).reshape(1, n)


def _tc_greedy_body(features_ref, out_ref):
    _, n, d = features_ref.shape
    iota_bn = lax.broadcasted_iota(jnp.int32, (TC_BB, n), 1)
    sal_rows = []
    for b in range(TC_BB):
        sal2 = features_ref[b, :, pl.ds(0, LANES)] ** 2
        for k in range(1, d // LANES):
            sal2 = sal2 + features_ref[b, :, pl.ds(k * LANES, LANES)] ** 2
        sal_rows.append(jnp.sqrt(jnp.sum(sal2, axis=1)).reshape(1, n))
    sal = jnp.concatenate(sal_rows, axis=0)
    denom = jnp.maximum(sal, EPS)
    mask = jnp.ones((TC_BB, n), dtype=jnp.float32)
    for r in range(N_SLOTS_K):
        ms = sal * mask
        mx = jnp.max(ms, axis=1, keepdims=True)
        idxs = jnp.min(jnp.where(ms == mx, iota_bn, n),
                       axis=1).astype(jnp.int32)
        sel_rows, snorm_rows = [], []
        for b in range(TC_BB):
            idx = idxs[b]
            sel = features_ref[b, pl.ds(idx, 1), :]
            out_ref[b, pl.ds(r, 1), :] = sel
            sel_rows.append(sel)
            snorm_rows.append(
                jnp.maximum(jnp.sqrt(jnp.sum(sel * sel)), EPS).reshape(1, 1))
        sels = jnp.concatenate(sel_rows, axis=0)
        snorm = jnp.concatenate(snorm_rows, axis=0)
        half_dots = []
        nh = n // 2
        for h in range(2):
            acc = (features_ref[:, pl.ds(h * nh, nh), pl.ds(0, LANES)]
                   * sels[:, None, 0:LANES])
            for k in range(1, d // LANES):
                acc = acc + (
                    features_ref[:, pl.ds(h * nh, nh), pl.ds(k * LANES, LANES)]
                    * sels[:, None, k * LANES:(k + 1) * LANES])
            half_dots.append(jnp.sum(acc, axis=2))
        dots = jnp.concatenate(half_dots, axis=1)
        sim = dots / (denom * snorm)
        mask = mask * (1.0 - jnp.clip(sim, 0.0, 1.0))


def _tc_part(features):
    b, n, d = features.shape
    return pl.pallas_call(
        _tc_greedy_body,
        grid=(b // TC_BB,),
        in_specs=[pl.BlockSpec((TC_BB, n, d), lambda i: (i, 0, 0))],
        out_specs=pl.BlockSpec((TC_BB, N_SLOTS_K, d), lambda i: (i, 0, 0)),
        out_shape=jax.ShapeDtypeStruct((b, N_SLOTS_K, d), features.dtype),
    )(features)


def _sc_part(features):
    b, n, d = features.shape
    mesh = plsc.VectorSubcoreMesh(core_axis_name="c", subcore_axis_name="s")
    f = functools.partial(
        pl.kernel,
        out_type=jax.ShapeDtypeStruct((b, N_SLOTS_K, d), features.dtype),
        mesh=mesh,
        compiler_params=pltpu.CompilerParams(
            needs_layout_passes=False, use_tc_tiling_on_sc=False),
        cost_estimate=pl.CostEstimate(
            flops=2 * b * n * d * (N_SLOTS_K + 1),
            bytes_accessed=4 * b * n * d,
            transcendentals=0,
        ),
        scratch_types=[
            pltpu.VMEM((ROWS_PER_TILE * d,), jnp.float32),
            pltpu.VMEM((d,), jnp.float32),
            pltpu.VMEM((ROWS_PER_TILE,), jnp.float32),
            pltpu.VMEM((L,), jnp.float32),
            pltpu.VMEM((L * L,), jnp.float32),
            pltpu.VMEM_SHARED((L * L,), jnp.float32),
            pltpu.VMEM_SHARED((d,), jnp.float32),
        ],
    )(_sc_greedy)
    return f(features.reshape(b, n * d))


def kernel(batch_size, features, fallback):
    del batch_size, fallback
    return _tc_part(features)


# speedup vs baseline: 1.3209x; 1.3209x over previous
import functools

import jax
import jax.numpy as jnp
from jax import lax
from jax.experimental import pallas as pl
from jax.experimental.pallas import tpu as pltpu
from jax.experimental.pallas import tpu_sc as plsc

N_SLOTS_K = 8
EPS = 1e-12
BIG = 1e9
L = 16
ROWS_PER_TILE = 64
GROUPS = ROWS_PER_TILE // L


def _rsqrt(x):
    i = plsc.bitcast(x, jnp.int32)
    seed = jnp.int32(0x5F3759DF) - lax.shift_right_logical(i, 1)
    y = plsc.bitcast(seed, jnp.float32)
    for _ in range(3):
        y = y * (1.5 - 0.5 * x * y * y)
    return y


def _sc_greedy(features_hbm, out_hbm, ft_v, sel_v, dots_v, cand_v,
               allcand_v, shared_cand, shared_sel):
    n_slots, d = N_SLOTS_K, out_hbm.shape[2]
    nchunk = d // L
    batches_per_core = features_hbm.shape[0] // 2
    core = lax.axis_index("c")
    sid = lax.axis_index("s")
    iota = lax.iota(jnp.int32, L)
    row_base = sid * ROWS_PER_TILE
    zero = jnp.zeros((L,), jnp.float32)
    gidx = [(row_base + g * L + iota).astype(jnp.float32) for g in range(GROUPS)]

    def streaming_pass(sel_chunks):
        def g_body(g, c):
            def j_body(j, dvec):
                off = (g * L + j) * d
                acc = zero
                for k in range(nchunk):
                    v = ft_v[pl.ds(off + k * L, L)]
                    acc = acc + v * (v if sel_chunks is None else sel_chunks[k])
                s = jnp.sum(acc)
                return jnp.where(iota == j, s, dvec)
            dvec = lax.fori_loop(0, L, j_body, zero, unroll=2)
            dots_v[pl.ds(g * L, L)] = dvec
            return c
        lax.fori_loop(0, GROUPS, g_body, 0)
        return [dots_v[pl.ds(g * L, L)] for g in range(GROUPS)]

    def run_batch(bi, carry):
        b = core * batches_per_core + bi
        pltpu.sync_copy(
            features_hbm.at[b, pl.ds(row_base * d, ROWS_PER_TILE * d)], ft_v)
        sal2 = streaming_pass(None)
        inv_sal = [jnp.minimum(_rsqrt(s), 1.0 / EPS) for s in sal2]
        mask = [jnp.ones((L,), jnp.float32) for _ in range(GROUPS)]

        for r in range(n_slots):
            ms = [sal2[g] * (mask[g] * mask[g]) for g in range(GROUPS)]
            mv = ms[0]
            for g in range(1, GROUPS):
                mv = jnp.maximum(mv, ms[g])
            m = jnp.max(mv)
            cidx = jnp.float32(BIG)
            for g in range(GROUPS):
                cidx = jnp.minimum(
                    cidx, jnp.min(jnp.where(ms[g] == m, gidx[g], BIG)))
            salsel = jnp.float32(-BIG)
            for g in range(GROUPS):
                salsel = jnp.maximum(
                    salsel, jnp.max(jnp.where(gidx[g] == cidx, sal2[g], -BIG)))
            cand = jnp.where(iota == 0, m,
                             jnp.where(iota == 1, cidx,
                                       jnp.where(iota == 2, salsel, 0.0)))
            cand_v[...] = cand
            pltpu.sync_copy(cand_v, shared_cand.at[pl.ds(sid * L, L)])
            plsc.subcore_barrier()
            pltpu.sync_copy(shared_cand, allcand_v)
            gm = jnp.float32(-BIG)
            widx = jnp.float32(BIG)
            wsal = jnp.float32(0.0)
            for t in range(L):
                row = allcand_v[pl.ds(t * L, L)]
                vt, it, st = row[0], row[1], row[2]
                better = (vt > gm) | ((vt == gm) & (it < widx))
                gm = jnp.where(better, vt, gm)
                widx = jnp.where(better, it, widx)
                wsal = jnp.where(better, st, wsal)
            widx_i = widx.astype(jnp.int32)
            wtile = widx_i // ROWS_PER_TILE
            @pl.when(wtile == sid)
            def _():
                pltpu.sync_copy(
                    ft_v.at[pl.ds((widx_i - row_base) * d, d)], shared_sel)
            plsc.subcore_barrier()
            pltpu.sync_copy(shared_sel, sel_v)

            @pl.when(sid == 0)
            def _():
                pltpu.sync_copy(sel_v, out_hbm.at[b, r])

            sel_chunks = [sel_v[pl.ds(k * L, L)] for k in range(nchunk)]
            dots = streaming_pass(sel_chunks)
            inv_sel = jnp.minimum(_rsqrt(jnp.full((L,), wsal)), 1.0 / EPS)
            for g in range(GROUPS):
                sim = dots[g] * inv_sal[g] * inv_sel
                mask[g] = mask[g] * (1.0 - jnp.clip(sim, 0.0, 1.0))
        return carry

    lax.fori_loop(0, batches_per_core, run_batch, 0)


LANES = 128
TC_BB = 8


def _tc_rowdot(a_ref, b_idx, other, n, d):
    acc = a_ref[b_idx, :, pl.ds(0, LANES)] * other[:, 0:LANES]
    for k in range(1, d // LANES):
        acc = acc + (a_ref[b_idx, :, pl.ds(k * LANES, LANES)]
                     * other[:, k * LANES:(k + 1) * LANES])
    return jnp.sum(acc, axis=1).reshape(1, n)


def _tc_greedy_body(features_ref, out_ref):
    _, n, d = features_ref.shape
    iota_bn = lax.broadcasted_iota(jnp.int32, (TC_BB, n), 1)
    sal_rows = []
    for b in range(TC_BB):
        sal2 = features_ref[b, :, pl.ds(0, LANES)] ** 2
        for k in range(1, d // LANES):
            sal2 = sal2 + features_ref[b, :, pl.ds(k * LANES, LANES)] ** 2
        sal_rows.append(jnp.sqrt(jnp.sum(sal2, axis=1)).reshape(1, n))
    sal = jnp.concatenate(sal_rows, axis=0)
    denom = jnp.maximum(sal, EPS)
    mask = jnp.ones((TC_BB, n), dtype=jnp.float32)
    for r in range(N_SLOTS_K):
        ms = sal * mask
        mx = jnp.max(ms, axis=1, keepdims=True)
        idxs = jnp.min(jnp.where(ms == mx, iota_bn, n),
                       axis=1).astype(jnp.int32)
        dot_rows, snorm_rows = [], []
        for b in range(TC_BB):
            idx = idxs[b]
            sel = features_ref[b, pl.ds(idx, 1), :]
            out_ref[b, pl.ds(r, 1), :] = sel
            dot_rows.append(_tc_rowdot(features_ref, b, sel, n, d))
            snorm_rows.append(
                jnp.maximum(jnp.sqrt(jnp.sum(sel * sel)), EPS).reshape(1, 1))
        dots = jnp.concatenate(dot_rows, axis=0)
        snorm = jnp.concatenate(snorm_rows, axis=0)
        sim = dots / (denom * snorm)
        mask = mask * (1.0 - jnp.clip(sim, 0.0, 1.0))


def _tc_part(features):
    b, n, d = features.shape
    return pl.pallas_call(
        _tc_greedy_body,
        grid=(b // TC_BB,),
        in_specs=[pl.BlockSpec((TC_BB, n, d), lambda i: (i, 0, 0))],
        out_specs=pl.BlockSpec((TC_BB, N_SLOTS_K, d), lambda i: (i, 0, 0)),
        out_shape=jax.ShapeDtypeStruct((b, N_SLOTS_K, d), features.dtype),
    )(features)


def _sc_part(features):
    b, n, d = features.shape
    mesh = plsc.VectorSubcoreMesh(core_axis_name="c", subcore_axis_name="s")
    f = functools.partial(
        pl.kernel,
        out_type=jax.ShapeDtypeStruct((b, N_SLOTS_K, d), features.dtype),
        mesh=mesh,
        compiler_params=pltpu.CompilerParams(
            needs_layout_passes=False, use_tc_tiling_on_sc=False),
        cost_estimate=pl.CostEstimate(
            flops=2 * b * n * d * (N_SLOTS_K + 1),
            bytes_accessed=4 * b * n * d,
            transcendentals=0,
        ),
        scratch_types=[
            pltpu.VMEM((ROWS_PER_TILE * d,), jnp.float32),
            pltpu.VMEM((d,), jnp.float32),
            pltpu.VMEM((ROWS_PER_TILE,), jnp.float32),
            pltpu.VMEM((L,), jnp.float32),
            pltpu.VMEM((L * L,), jnp.float32),
            pltpu.VMEM_SHARED((L * L,), jnp.float32),
            pltpu.VMEM_SHARED((d,), jnp.float32),
        ],
    )(_sc_greedy)
    return f(features.reshape(b, n * d))


SC_B = 8


def kernel(batch_size, features, fallback):
    del batch_size, fallback
    return _tc_part(features)


# speedup vs baseline: 1.3255x; 1.0035x over previous
import functools

import jax
import jax.numpy as jnp
from jax import lax
from jax.experimental import pallas as pl
from jax.experimental.pallas import tpu as pltpu
from jax.experimental.pallas import tpu_sc as plsc

N_SLOTS_K = 8
EPS = 1e-12
BIG = 1e9
L = 16
ROWS_PER_TILE = 64
GROUPS = ROWS_PER_TILE // L


def _rsqrt(x):
    i = plsc.bitcast(x, jnp.int32)
    seed = jnp.int32(0x5F3759DF) - lax.shift_right_logical(i, 1)
    y = plsc.bitcast(seed, jnp.float32)
    for _ in range(3):
        y = y * (1.5 - 0.5 * x * y * y)
    return y


def _sc_greedy(features_hbm, out_hbm, ft_v, sel_v, dots_v, cand_v,
               allcand_v, shared_cand, shared_sel):
    n_slots, d = N_SLOTS_K, out_hbm.shape[2]
    nchunk = d // L
    batches_per_core = features_hbm.shape[0] // 2
    core = lax.axis_index("c")
    sid = lax.axis_index("s")
    iota = lax.iota(jnp.int32, L)
    row_base = sid * ROWS_PER_TILE
    zero = jnp.zeros((L,), jnp.float32)
    gidx = [(row_base + g * L + iota).astype(jnp.float32) for g in range(GROUPS)]

    def streaming_pass(sel_chunks):
        def g_body(g, c):
            def j_body(j, dvec):
                off = (g * L + j) * d
                acc = zero
                for k in range(nchunk):
                    v = ft_v[pl.ds(off + k * L, L)]
                    acc = acc + v * (v if sel_chunks is None else sel_chunks[k])
                s = jnp.sum(acc)
                return jnp.where(iota == j, s, dvec)
            dvec = lax.fori_loop(0, L, j_body, zero, unroll=2)
            dots_v[pl.ds(g * L, L)] = dvec
            return c
        lax.fori_loop(0, GROUPS, g_body, 0)
        return [dots_v[pl.ds(g * L, L)] for g in range(GROUPS)]

    def run_batch(bi, carry):
        b = core * batches_per_core + bi
        pltpu.sync_copy(
            features_hbm.at[b, pl.ds(row_base * d, ROWS_PER_TILE * d)], ft_v)
        sal2 = streaming_pass(None)
        inv_sal = [jnp.minimum(_rsqrt(s), 1.0 / EPS) for s in sal2]
        mask = [jnp.ones((L,), jnp.float32) for _ in range(GROUPS)]

        for r in range(n_slots):
            ms = [sal2[g] * (mask[g] * mask[g]) for g in range(GROUPS)]
            mv = ms[0]
            for g in range(1, GROUPS):
                mv = jnp.maximum(mv, ms[g])
            m = jnp.max(mv)
            cidx = jnp.float32(BIG)
            for g in range(GROUPS):
                cidx = jnp.minimum(
                    cidx, jnp.min(jnp.where(ms[g] == m, gidx[g], BIG)))
            salsel = jnp.float32(-BIG)
            for g in range(GROUPS):
                salsel = jnp.maximum(
                    salsel, jnp.max(jnp.where(gidx[g] == cidx, sal2[g], -BIG)))
            cand = jnp.where(iota == 0, m,
                             jnp.where(iota == 1, cidx,
                                       jnp.where(iota == 2, salsel, 0.0)))
            cand_v[...] = cand
            pltpu.sync_copy(cand_v, shared_cand.at[pl.ds(sid * L, L)])
            plsc.subcore_barrier()
            pltpu.sync_copy(shared_cand, allcand_v)
            gm = jnp.float32(-BIG)
            widx = jnp.float32(BIG)
            wsal = jnp.float32(0.0)
            for t in range(L):
                row = allcand_v[pl.ds(t * L, L)]
                vt, it, st = row[0], row[1], row[2]
                better = (vt > gm) | ((vt == gm) & (it < widx))
                gm = jnp.where(better, vt, gm)
                widx = jnp.where(better, it, widx)
                wsal = jnp.where(better, st, wsal)
            widx_i = widx.astype(jnp.int32)
            wtile = widx_i // ROWS_PER_TILE
            @pl.when(wtile == sid)
            def _():
                pltpu.sync_copy(
                    ft_v.at[pl.ds((widx_i - row_base) * d, d)], shared_sel)
            plsc.subcore_barrier()
            pltpu.sync_copy(shared_sel, sel_v)

            @pl.when(sid == 0)
            def _():
                pltpu.sync_copy(sel_v, out_hbm.at[b, r])

            sel_chunks = [sel_v[pl.ds(k * L, L)] for k in range(nchunk)]
            dots = streaming_pass(sel_chunks)
            inv_sel = jnp.minimum(_rsqrt(jnp.full((L,), wsal)), 1.0 / EPS)
            for g in range(GROUPS):
                sim = dots[g] * inv_sal[g] * inv_sel
                mask[g] = mask[g] * (1.0 - jnp.clip(sim, 0.0, 1.0))
        return carry

    lax.fori_loop(0, batches_per_core, run_batch, 0)


LANES = 128
TC_BB = 8


def _tc_rowdot(a_ref, b_idx, other, n, d):
    acc = a_ref[b_idx, :, pl.ds(0, LANES)] * other[:, 0:LANES]
    for k in range(1, d // LANES):
        acc = acc + (a_ref[b_idx, :, pl.ds(k * LANES, LANES)]
                     * other[:, k * LANES:(k + 1) * LANES])
    return jnp.sum(acc, axis=1).reshape(1, n)


def _tc_greedy_body(features_ref, out_ref):
    _, n, d = features_ref.shape
    iota_bn = lax.broadcasted_iota(jnp.int32, (TC_BB, n), 1)
    sal_rows = []
    for b in range(TC_BB):
        sal2 = features_ref[b, :, pl.ds(0, LANES)] ** 2
        for k in range(1, d // LANES):
            sal2 = sal2 + features_ref[b, :, pl.ds(k * LANES, LANES)] ** 2
        sal_rows.append(jnp.sqrt(jnp.sum(sal2, axis=1)).reshape(1, n))
    sal = jnp.concatenate(sal_rows, axis=0)
    denom = jnp.maximum(sal, EPS)
    mask = jnp.ones((TC_BB, n), dtype=jnp.float32)
    for r in range(N_SLOTS_K):
        ms = sal * mask
        mx = jnp.max(ms, axis=1, keepdims=True)
        idxs = jnp.min(jnp.where(ms == mx, iota_bn, n),
                       axis=1).astype(jnp.int32)
        dot_rows, snorm_rows = [], []
        for b in range(TC_BB):
            idx = idxs[b]
            sel = features_ref[b, pl.ds(idx, 1), :]
            out_ref[b, pl.ds(r, 1), :] = sel
            dot_rows.append(_tc_rowdot(features_ref, b, sel, n, d))
            snorm_rows.append(
                jnp.maximum(jnp.sqrt(jnp.sum(sel * sel)), EPS).reshape(1, 1))
        dots = jnp.concatenate(dot_rows, axis=0)
        snorm = jnp.concatenate(snorm_rows, axis=0)
        sim = dots / (denom * snorm)
        mask = mask * (1.0 - jnp.clip(sim, 0.0, 1.0))


def _tc_part(features):
    b, n, d = features.shape
    return pl.pallas_call(
        _tc_greedy_body,
        grid=(b // TC_BB,),
        in_specs=[pl.BlockSpec((TC_BB, n, d), lambda i: (i, 0, 0))],
        out_specs=pl.BlockSpec((TC_BB, N_SLOTS_K, d), lambda i: (i, 0, 0)),
        out_shape=jax.ShapeDtypeStruct((b, N_SLOTS_K, d), features.dtype),
        compiler_params=pltpu.CompilerParams(
            dimension_semantics=("parallel",)),
    )(features)


def _sc_part(features):
    b, n, d = features.shape
    mesh = plsc.VectorSubcoreMesh(core_axis_name="c", subcore_axis_name="s")
    f = functools.partial(
        pl.kernel,
        out_type=jax.ShapeDtypeStruct((b, N_SLOTS_K, d), features.dtype),
        mesh=mesh,
        compiler_params=pltpu.CompilerParams(
            needs_layout_passes=False, use_tc_tiling_on_sc=False),
        cost_estimate=pl.CostEstimate(
            flops=2 * b * n * d * (N_SLOTS_K + 1),
            bytes_accessed=4 * b * n * d,
            transcendentals=0,
        ),
        scratch_types=[
            pltpu.VMEM((ROWS_PER_TILE * d,), jnp.float32),
            pltpu.VMEM((d,), jnp.float32),
            pltpu.VMEM((ROWS_PER_TILE,), jnp.float32),
            pltpu.VMEM((L,), jnp.float32),
            pltpu.VMEM((L * L,), jnp.float32),
            pltpu.VMEM_SHARED((L * L,), jnp.float32),
            pltpu.VMEM_SHARED((d,), jnp.float32),
        ],
    )(_sc_greedy)
    return f(features.reshape(b, n * d))


def kernel(batch_size, features, fallback):
    del batch_size, fallback
    return _tc_part(features)
